# Initial kernel scaffold; baseline (speedup 1.0000x reference)
import jax
import jax.numpy as jnp
from jax.experimental import pallas as pl


def kernel(node_feat, edge_feat, edge_index, graph_ids, Wn, bn, We, be, Wc0, bc0, Wc1, bc1, Wc2, bc2, Wfp, bfp):
    raise NotImplementedError("write your pallas kernel here")


# speedup vs baseline: 5.7745x; 5.7745x over previous
import functools
from functools import partial

import jax
import jax.numpy as jnp
from jax import lax
from jax.experimental import pallas as pl
from jax.experimental.pallas import tpu as pltpu
from jax.experimental.pallas import tpu_sc as plsc

N = 10000
E = 320000
D = 128
DE = 16
G = 64

NC = 2
NS = 16
NW = NC * NS
EPW = E // NW
K = 100
NCHUNK = EPW // K
NP = 10240
RPS = NP // NS
ZR = 80
NZ = RPS // ZR

_mesh = plsc.VectorSubcoreMesh(
    core_axis_name="c", subcore_axis_name="s", num_cores=NC, num_subcores=NS)


def _zero_rows(zbuf, nrows, ncols):
  z = jnp.zeros((16,), jnp.float32)

  def body(i, _):
    for k in range(ncols // 16):
      zbuf[i, pl.ds(k * 16, 16)] = z
    return 0

  lax.fori_loop(0, nrows, body, 0)


def _ones_rows(obuf, nrows, ncols):
  o = jnp.ones((16,), jnp.float32)

  def body(i, _):
    for k in range(ncols // 16):
      obuf[i, pl.ds(k * 16, 16)] = o
    return 0

  lax.fori_loop(0, nrows, body, 0)


@partial(
    pl.kernel,
    out_type=(
        jax.ShapeDtypeStruct((NC, NP, DE), jnp.float32),
        jax.ShapeDtypeStruct((NC, NP, DE), jnp.float32),
    ),
    mesh=_mesh,
    compiler_params=pltpu.CompilerParams(use_tc_tiling_on_sc=False),
    scratch_types=[
        pltpu.VMEM((NCHUNK, K), jnp.int32),
        pltpu.VMEM((K, DE), jnp.float32),
        pltpu.VMEM((K, DE), jnp.float32),
        pltpu.VMEM_SHARED((NP, DE), jnp.float32),
        pltpu.VMEM_SHARED((NP, DE), jnp.float32),
        pltpu.SemaphoreType.DMA,
    ],
)
def _edge_pool_sc(ef_hbm, dst_hbm, seg_hbm, cnt_hbm, idx_v, rows_v, ones_v,
                  acc_sh, cnt_sh, sem):
  c = lax.axis_index("c")
  s = lax.axis_index("s")
  wid = c * NS + s

  _zero_rows(rows_v, ZR, DE)
  _ones_rows(ones_v, K, DE)
  zsrc = rows_v.at[pl.ds(0, ZR)]
  for t in range(NZ):
    pltpu.sync_copy(zsrc, acc_sh.at[pl.ds(s * RPS + t * ZR, ZR)])
    pltpu.sync_copy(zsrc, cnt_sh.at[pl.ds(s * RPS + t * ZR, ZR)])
  plsc.subcore_barrier()

  pltpu.sync_copy(dst_hbm.at[wid], idx_v)

  def body(j, _):
    pltpu.async_copy(ef_hbm.at[wid, j], rows_v, sem).wait()
    pltpu.sync_copy(rows_v, acc_sh.at[idx_v.at[j]], add=True)
    pltpu.sync_copy(ones_v, cnt_sh.at[idx_v.at[j]], add=True)
    return 0

  lax.fori_loop(0, NCHUNK, body, 0)
  plsc.subcore_barrier()

  pltpu.sync_copy(acc_sh.at[pl.ds(s * RPS, RPS)],
                  seg_hbm.at[c, pl.ds(s * RPS, RPS)])
  pltpu.sync_copy(cnt_sh.at[pl.ds(s * RPS, RPS)],
                  cnt_hbm.at[c, pl.ds(s * RPS, RPS)])


@partial(
    pl.kernel,
    out_type=jax.ShapeDtypeStruct((NC, NP, D), jnp.float32),
    mesh=_mesh,
    scratch_types=[
        pltpu.VMEM((NCHUNK, K), jnp.int32),
        pltpu.VMEM((NCHUNK, K), jnp.int32),
        pltpu.VMEM((K, D), jnp.float32),
        pltpu.VMEM_SHARED((NP, D), jnp.float32),
        pltpu.SemaphoreType.DMA,
    ],
)
def _neigh_pool_sc(cur_hbm, src_hbm, dst_hbm, out_hbm, src_v, dst_v, rows_v,
                   acc_sh, sem):
  c = lax.axis_index("c")
  s = lax.axis_index("s")
  wid = c * NS + s

  _zero_rows(rows_v, ZR, D)
  zsrc = rows_v.at[pl.ds(0, ZR)]
  for t in range(NZ):
    pltpu.sync_copy(zsrc, acc_sh.at[pl.ds(s * RPS + t * ZR, ZR)])
  plsc.subcore_barrier()

  pltpu.sync_copy(src_hbm.at[wid], src_v)
  pltpu.sync_copy(dst_hbm.at[wid], dst_v)

  def body(j, _):
    pltpu.async_copy(cur_hbm.at[src_v.at[j]], rows_v, sem).wait()
    pltpu.sync_copy(rows_v, acc_sh.at[dst_v.at[j]], add=True)
    return 0

  lax.fori_loop(0, NCHUNK, body, 0)
  plsc.subcore_barrier()

  pltpu.sync_copy(acc_sh.at[pl.ds(s * RPS, RPS)],
                  out_hbm.at[c, pl.ds(s * RPS, RPS)])


NB = 10
BR = N // NB


def _msg_init_tc(nf, seg, cnt, Wn, We, bn, be, im_out, cur_out):
  s = seg[0] + seg[1]
  deg = cnt[0][:, 0:1] + cnt[1][:, 0:1]
  im = (jnp.dot(nf[...], Wn[...], preferred_element_type=jnp.float32)
        + jnp.dot(s, We[...], preferred_element_type=jnp.float32)
        + bn[...] + deg * be[...])
  im_out[...] = im
  cur_out[...] = jnp.maximum(im, 0.0)


def _round_tc(p, im, Wc, bc, cur_out):
  npool = p[0] + p[1]
  nl = jnp.dot(npool, Wc[...], preferred_element_type=jnp.float32)
  cur_out[...] = jnp.maximum(nl + bc[...] + im[...], 0.0)


def _final_tc(cur, gids, Wfp, bfp, y_out):
  i = pl.program_id(0)
  r = jnp.maximum(
      jnp.dot(cur[...], Wfp[...], preferred_element_type=jnp.float32)
      + bfp[...], 0.0)
  ids = gids[0]
  onehot = (lax.broadcasted_iota(jnp.int32, (G, BR), 0) == ids).astype(
      jnp.float32)
  contrib = jnp.dot(onehot, r, preferred_element_type=jnp.float32)

  @pl.when(i == 0)
  def _():
    y_out[...] = jnp.zeros_like(y_out)

  y_out[...] += contrib


def kernel(node_feat, edge_feat, edge_index, graph_ids, Wn, bn, We, be,
           Wc0, bc0, Wc1, bc1, Wc2, bc2, Wfp, bfp):
  src = edge_index[0].reshape(NW, NCHUNK, K)
  dst = edge_index[1].reshape(NW, NCHUNK, K)
  ef = edge_feat.reshape(NW, NCHUNK, K, DE)
  gids3 = graph_ids.reshape(NB, 1, BR)
  bn2 = bn.reshape(1, D)
  be2 = be.reshape(1, D)
  bfp2 = bfp.reshape(1, D)

  seg16, cnt16 = _edge_pool_sc(ef, dst)

  row_spec = pl.BlockSpec((BR, D), lambda i: (i, 0))
  full2 = lambda shape: pl.BlockSpec(shape, lambda i: tuple(0 for _ in shape))

  im, cur = pl.pallas_call(
      _msg_init_tc,
      grid=(NB,),
      in_specs=[
          row_spec,
          pl.BlockSpec((NC, BR, DE), lambda i: (0, i, 0)),
          pl.BlockSpec((NC, BR, DE), lambda i: (0, i, 0)),
          full2((D, D)), full2((DE, D)), full2((1, D)), full2((1, D)),
      ],
      out_specs=[row_spec, row_spec],
      out_shape=[
          jax.ShapeDtypeStruct((N, D), jnp.float32),
          jax.ShapeDtypeStruct((N, D), jnp.float32),
      ],
  )(node_feat, seg16, cnt16, Wn, We, bn2, be2)

  round_call = pl.pallas_call(
      _round_tc,
      grid=(NB,),
      in_specs=[
          pl.BlockSpec((NC, BR, D), lambda i: (0, i, 0)),
          row_spec,
          full2((D, D)), full2((1, D)),
      ],
      out_specs=row_spec,
      out_shape=jax.ShapeDtypeStruct((N, D), jnp.float32),
  )

  for Wc, bc in ((Wc0, bc0), (Wc1, bc1), (Wc2, bc2)):
    partials = _neigh_pool_sc(cur, src, dst)
    cur = round_call(partials, im, Wc, bc.reshape(1, D))

  y = pl.pallas_call(
      _final_tc,
      grid=(NB,),
      in_specs=[
          row_spec,
          pl.BlockSpec((1, 1, BR), lambda i: (i, 0, 0)),
          full2((D, D)), full2((1, D)),
      ],
      out_specs=pl.BlockSpec((G, D), lambda i: (0, 0)),
      out_shape=jax.ShapeDtypeStruct((G, D), jnp.float32),
  )(cur, gids3, Wfp, bfp2)

  return y


# speedup vs baseline: 6.8064x; 1.1787x over previous
import functools
from functools import partial

import jax
import jax.numpy as jnp
from jax import lax
from jax.experimental import pallas as pl
from jax.experimental.pallas import tpu as pltpu
from jax.experimental.pallas import tpu_sc as plsc

N = 10000
E = 320000
D = 128
DE = 16
G = 64

NC = 2
NS = 16
NW = NC * NS
EPW = E // NW
KA = 100
NCHA = EPW // KA
KB = 50
NCHB = EPW // KB
WS = 8
NWIN = NCHB // WS
NP = 10240
RPS = NP // NS
ZR = 80
NZ = RPS // ZR
ZRB = 40
NZB = RPS // ZRB

_mesh = plsc.VectorSubcoreMesh(
    core_axis_name="c", subcore_axis_name="s", num_cores=NC, num_subcores=NS)


def _zero_rows(zbuf, nrows, ncols):
  z = jnp.zeros((16,), jnp.float32)

  def body(i, _):
    for k in range(ncols // 16):
      zbuf[i, pl.ds(k * 16, 16)] = z
    return 0

  lax.fori_loop(0, nrows, body, 0)


def _ones_rows(obuf, nrows, ncols):
  o = jnp.ones((16,), jnp.float32)

  def body(i, _):
    for k in range(ncols // 16):
      obuf[i, pl.ds(k * 16, 16)] = o
    return 0

  lax.fori_loop(0, nrows, body, 0)


@partial(
    pl.kernel,
    out_type=(
        jax.ShapeDtypeStruct((NC, NP, DE), jnp.float32),
        jax.ShapeDtypeStruct((NC, NP, DE), jnp.float32),
    ),
    mesh=_mesh,
    compiler_params=pltpu.CompilerParams(use_tc_tiling_on_sc=False),
    scratch_types=[
        pltpu.VMEM((NCHA, KA), jnp.int32),
        pltpu.VMEM((KA, DE), jnp.float32),
        pltpu.VMEM((KA, DE), jnp.float32),
        pltpu.VMEM_SHARED((NP, DE), jnp.float32),
        pltpu.VMEM_SHARED((NP, DE), jnp.float32),
        pltpu.SemaphoreType.DMA,
    ],
)
def _edge_pool_sc(ef_hbm, dst_hbm, seg_hbm, cnt_hbm, idx_v, rows_v, ones_v,
                  acc_sh, cnt_sh, sem):
  c = lax.axis_index("c")
  s = lax.axis_index("s")
  wid = c * NS + s

  _zero_rows(rows_v, ZR, DE)
  _ones_rows(ones_v, KA, DE)
  zsrc = rows_v.at[pl.ds(0, ZR)]
  for t in range(NZ):
    pltpu.sync_copy(zsrc, acc_sh.at[pl.ds(s * RPS + t * ZR, ZR)])
    pltpu.sync_copy(zsrc, cnt_sh.at[pl.ds(s * RPS + t * ZR, ZR)])
  plsc.subcore_barrier()

  pltpu.sync_copy(dst_hbm.at[wid], idx_v)

  def body(j, _):
    pltpu.async_copy(ef_hbm.at[wid, j], rows_v, sem).wait()
    pltpu.sync_copy(rows_v, acc_sh.at[idx_v.at[j]], add=True)
    pltpu.sync_copy(ones_v, cnt_sh.at[idx_v.at[j]], add=True)
    return 0

  lax.fori_loop(0, NCHA, body, 0)
  plsc.subcore_barrier()

  pltpu.sync_copy(acc_sh.at[pl.ds(s * RPS, RPS)],
                  seg_hbm.at[c, pl.ds(s * RPS, RPS)])
  pltpu.sync_copy(cnt_sh.at[pl.ds(s * RPS, RPS)],
                  cnt_hbm.at[c, pl.ds(s * RPS, RPS)])


@partial(
    pl.kernel,
    out_type=jax.ShapeDtypeStruct((NC, NP, D), jnp.float32),
    mesh=_mesh,
    scratch_types=[
        pltpu.VMEM((WS, KB), jnp.int32),
        pltpu.VMEM((WS, KB), jnp.int32),
        pltpu.VMEM((WS, KB), jnp.int32),
        pltpu.VMEM((WS, KB), jnp.int32),
        pltpu.VMEM((KB, D), jnp.float32),
        pltpu.VMEM((KB, D), jnp.float32),
        pltpu.VMEM_SHARED((NP, D), jnp.float32),
        pltpu.SemaphoreType.DMA,
        pltpu.SemaphoreType.DMA,
        pltpu.SemaphoreType.DMA,
        pltpu.SemaphoreType.DMA,
    ],
)
def _neigh_pool_sc(cur_hbm, src_hbm, dst_hbm, out_hbm, sidx0, sidx1, didx0,
                   didx1, rows0, rows1, acc_sh, gsem0, gsem1, isem0, isem1):
  c = lax.axis_index("c")
  s = lax.axis_index("s")
  wid = c * NS + s
  rows = (rows0, rows1)
  gsems = (gsem0, gsem1)
  sidx = (sidx0, sidx1)
  didx = (didx0, didx1)
  isems = (isem0, isem1)

  _zero_rows(rows0, ZRB, D)
  zsrc = rows0.at[pl.ds(0, ZRB)]
  for t in range(NZB):
    pltpu.sync_copy(zsrc, acc_sh.at[pl.ds(s * RPS + t * ZRB, ZRB)])
  plsc.subcore_barrier()

  def win_slice(ref, w):
    off = pl.multiple_of(w * WS, WS)
    return ref.at[wid, pl.ds(off, WS)]

  pltpu.sync_copy(win_slice(src_hbm, 0), sidx0)
  pltpu.sync_copy(win_slice(dst_hbm, 0), didx0)
  pltpu.async_copy(win_slice(src_hbm, 1), sidx1, isem1)
  pltpu.async_copy(win_slice(dst_hbm, 1), didx1, isem1)
  pltpu.async_copy(cur_hbm.at[sidx0.at[0]], rows0, gsem0)

  def do_window(w, wb):
    for t in range(WS):
      b = t % 2
      if t == WS - 1:
        nxt_first = (w + 1) * WS

        @pl.when(nxt_first < NCHB)
        def _():
          pltpu.make_async_copy(win_slice(src_hbm, w + 1), sidx[1 - wb],
                                isems[1 - wb]).wait()
          pltpu.make_async_copy(win_slice(dst_hbm, w + 1), didx[1 - wb],
                                isems[1 - wb]).wait()
          pltpu.async_copy(cur_hbm.at[sidx[1 - wb].at[0]], rows[1 - b],
                           gsems[1 - b])
      else:
        pltpu.async_copy(cur_hbm.at[sidx[wb].at[t + 1]], rows[1 - b],
                         gsems[1 - b])
      pltpu.make_async_copy(cur_hbm.at[sidx[wb].at[t]], rows[b],
                            gsems[b]).wait()
      pltpu.sync_copy(rows[b], acc_sh.at[didx[wb].at[t]], add=True)

    @pl.when(w + 2 < NWIN)
    def _():
      pltpu.async_copy(win_slice(src_hbm, w + 2), sidx[wb], isems[wb])
      pltpu.async_copy(win_slice(dst_hbm, w + 2), didx[wb], isems[wb])

  def wpair(i, _):
    do_window(i * 2, 0)
    do_window(i * 2 + 1, 1)
    return 0

  lax.fori_loop(0, NWIN // 2, wpair, 0)
  do_window(NWIN - 1, 0)
  plsc.subcore_barrier()

  pltpu.sync_copy(acc_sh.at[pl.ds(s * RPS, RPS)],
                  out_hbm.at[c, pl.ds(s * RPS, RPS)])


NB = 10
BR = N // NB


def _msg_init_tc(nf, seg, cnt, Wn, We, bn, be, im_out, cur_out):
  s = seg[0] + seg[1]
  deg = cnt[0][:, 0:1] + cnt[1][:, 0:1]
  im = (jnp.dot(nf[...], Wn[...], preferred_element_type=jnp.float32)
        + jnp.dot(s, We[...], preferred_element_type=jnp.float32)
        + bn[...] + deg * be[...])
  im_out[...] = im
  cur_out[...] = jnp.maximum(im, 0.0)


def _round_tc(p, im, Wc, bc, cur_out):
  npool = p[0] + p[1]
  nl = jnp.dot(npool, Wc[...], preferred_element_type=jnp.float32)
  cur_out[...] = jnp.maximum(nl + bc[...] + im[...], 0.0)


def _final_tc(cur, gids, Wfp, bfp, y_out):
  i = pl.program_id(0)
  r = jnp.maximum(
      jnp.dot(cur[...], Wfp[...], preferred_element_type=jnp.float32)
      + bfp[...], 0.0)
  ids = gids[0]
  onehot = (lax.broadcasted_iota(jnp.int32, (G, BR), 0) == ids).astype(
      jnp.float32)
  contrib = jnp.dot(onehot, r, preferred_element_type=jnp.float32)

  @pl.when(i == 0)
  def _():
    y_out[...] = jnp.zeros_like(y_out)

  y_out[...] += contrib


def kernel(node_feat, edge_feat, edge_index, graph_ids, Wn, bn, We, be,
           Wc0, bc0, Wc1, bc1, Wc2, bc2, Wfp, bfp):
  src = edge_index[0].reshape(NW, NCHB, KB)
  dst = edge_index[1].reshape(NW, NCHB, KB)
  dst_a = edge_index[1].reshape(NW, NCHA, KA)
  ef = edge_feat.reshape(NW, NCHA, KA, DE)
  gids3 = graph_ids.reshape(NB, 1, BR)
  bn2 = bn.reshape(1, D)
  be2 = be.reshape(1, D)
  bfp2 = bfp.reshape(1, D)

  seg16, cnt16 = _edge_pool_sc(ef, dst_a)

  row_spec = pl.BlockSpec((BR, D), lambda i: (i, 0))
  full2 = lambda shape: pl.BlockSpec(shape, lambda i: tuple(0 for _ in shape))

  im, cur = pl.pallas_call(
      _msg_init_tc,
      grid=(NB,),
      in_specs=[
          row_spec,
          pl.BlockSpec((NC, BR, DE), lambda i: (0, i, 0)),
          pl.BlockSpec((NC, BR, DE), lambda i: (0, i, 0)),
          full2((D, D)), full2((DE, D)), full2((1, D)), full2((1, D)),
      ],
      out_specs=[row_spec, row_spec],
      out_shape=[
          jax.ShapeDtypeStruct((N, D), jnp.float32),
          jax.ShapeDtypeStruct((N, D), jnp.float32),
      ],
  )(node_feat, seg16, cnt16, Wn, We, bn2, be2)

  round_call = pl.pallas_call(
      _round_tc,
      grid=(NB,),
      in_specs=[
          pl.BlockSpec((NC, BR, D), lambda i: (0, i, 0)),
          row_spec,
          full2((D, D)), full2((1, D)),
      ],
      out_specs=row_spec,
      out_shape=jax.ShapeDtypeStruct((N, D), jnp.float32),
  )

  for Wc, bc in ((Wc0, bc0), (Wc1, bc1), (Wc2, bc2)):
    partials = _neigh_pool_sc(cur, src, dst)
    cur = round_call(partials, im, Wc, bc.reshape(1, D))

  y = pl.pallas_call(
      _final_tc,
      grid=(NB,),
      in_specs=[
          row_spec,
          pl.BlockSpec((1, 1, BR), lambda i: (i, 0, 0)),
          full2((D, D)), full2((1, D)),
      ],
      out_specs=pl.BlockSpec((G, D), lambda i: (0, 0)),
      out_shape=jax.ShapeDtypeStruct((G, D), jnp.float32),
  )(cur, gids3, Wfp, bfp2)

  return y


# speedup vs baseline: 8.2452x; 1.2114x over previous
import functools
from functools import partial

import jax
import jax.numpy as jnp
from jax import lax
from jax.experimental import pallas as pl
from jax.experimental.pallas import tpu as pltpu
from jax.experimental.pallas import tpu_sc as plsc

N = 10000
E = 320000
D = 128
DE = 16
G = 64

NC = 2
NS = 16
NW = NC * NS
EPW = E // NW
KA = 100
NCHA = EPW // KA
KB = 50
NCHB = EPW // KB
WS = 8
NWIN = NCHB // WS
NP = 10240
RPS = NP // NS
ZR = 80
NZ = RPS // ZR
ZRB = 40
NZB = RPS // ZRB

_mesh = plsc.VectorSubcoreMesh(
    core_axis_name="c", subcore_axis_name="s", num_cores=NC, num_subcores=NS)


def _zero_rows(zbuf, nrows, ncols):
  z = jnp.zeros((16,), jnp.float32)

  def body(i, _):
    for k in range(ncols // 16):
      zbuf[i, pl.ds(k * 16, 16)] = z
    return 0

  lax.fori_loop(0, nrows, body, 0)


def _ones_rows(obuf, nrows, ncols):
  o = jnp.ones((16,), jnp.float32)

  def body(i, _):
    for k in range(ncols // 16):
      obuf[i, pl.ds(k * 16, 16)] = o
    return 0

  lax.fori_loop(0, nrows, body, 0)


DA = 2 * DE


@partial(
    pl.kernel,
    out_type=jax.ShapeDtypeStruct((NC, NP, DA), jnp.float32),
    mesh=_mesh,
    compiler_params=pltpu.CompilerParams(use_tc_tiling_on_sc=False),
    scratch_types=[
        pltpu.VMEM((NCHA, KA), jnp.int32),
        pltpu.VMEM((KA, DA), jnp.float32),
        pltpu.VMEM((KA, DA), jnp.float32),
        pltpu.VMEM((KA, DA), jnp.float32),
        pltpu.VMEM_SHARED((NP, DA), jnp.float32),
        pltpu.SemaphoreType.DMA,
        pltpu.SemaphoreType.DMA,
        pltpu.SemaphoreType.DMA,
    ],
)
def _edge_pool_sc(ef_hbm, dst_hbm, out_hbm, idx_v, r0, r1, r2, acc_sh,
                  g0, g1, g2):
  c = lax.axis_index("c")
  s = lax.axis_index("s")
  wid = c * NS + s
  rows = (r0, r1, r2)
  gsems = (g0, g1, g2)

  _zero_rows(r0, ZR, DA)
  zsrc = r0.at[pl.ds(0, ZR)]
  for t in range(NZ):
    pltpu.sync_copy(zsrc, acc_sh.at[pl.ds(s * RPS + t * ZR, ZR)])
  plsc.subcore_barrier()

  onev = jnp.where(lax.iota(jnp.int32, 16) == 0, 1.0, 0.0).astype(jnp.float32)

  def initrows(i, _):
    for rb in rows:
      rb[i, pl.ds(DE, 16)] = onev
    return 0

  lax.fori_loop(0, KA, initrows, 0)

  pltpu.sync_copy(dst_hbm.at[wid], idx_v)

  def feat_dst(rb):
    return rb.at[:, pl.ds(0, DE)]

  pltpu.async_copy(ef_hbm.at[wid, 0], feat_dst(r0), g0)
  pltpu.async_copy(ef_hbm.at[wid, 1], feat_dst(r1), g1)

  def body(i, _):
    for b in range(3):
      j = i * 3 + b

      @pl.when(j + 2 < NCHA)
      def _():
        pltpu.async_copy(ef_hbm.at[wid, j + 2], feat_dst(rows[(b + 2) % 3]),
                         gsems[(b + 2) % 3])

      pltpu.make_async_copy(ef_hbm.at[wid, j], feat_dst(rows[b]),
                            gsems[b]).wait()
      pltpu.sync_copy(rows[b], acc_sh.at[idx_v.at[j]], add=True)
    return 0

  lax.fori_loop(0, NCHA // 3, body, 0)
  jt = NCHA - 1
  pltpu.make_async_copy(ef_hbm.at[wid, jt], feat_dst(r0), g0).wait()
  pltpu.sync_copy(r0, acc_sh.at[idx_v.at[jt]], add=True)
  plsc.subcore_barrier()

  pltpu.sync_copy(acc_sh.at[pl.ds(s * RPS, RPS)],
                  out_hbm.at[c, pl.ds(s * RPS, RPS)])


@partial(
    pl.kernel,
    out_type=jax.ShapeDtypeStruct((NC, NP, D), jnp.float32),
    mesh=_mesh,
    scratch_types=[
        pltpu.VMEM((WS, KB), jnp.int32),
        pltpu.VMEM((WS, KB), jnp.int32),
        pltpu.VMEM((WS, KB), jnp.int32),
        pltpu.VMEM((WS, KB), jnp.int32),
        pltpu.VMEM((KB, D), jnp.float32),
        pltpu.VMEM((KB, D), jnp.float32),
        pltpu.VMEM((KB, D), jnp.float32),
        pltpu.VMEM_SHARED((NP, D), jnp.float32),
        pltpu.SemaphoreType.DMA,
        pltpu.SemaphoreType.DMA,
        pltpu.SemaphoreType.DMA,
        pltpu.SemaphoreType.DMA,
        pltpu.SemaphoreType.DMA,
    ],
)
def _neigh_pool_sc(cur_hbm, src_hbm, dst_hbm, out_hbm, sidx0, sidx1, didx0,
                   didx1, rows0, rows1, rows2, acc_sh, gsem0, gsem1, gsem2,
                   isem0, isem1):
  c = lax.axis_index("c")
  s = lax.axis_index("s")
  wid = c * NS + s
  rows = (rows0, rows1, rows2)
  gsems = (gsem0, gsem1, gsem2)
  sidx = (sidx0, sidx1)
  didx = (didx0, didx1)
  isems = (isem0, isem1)

  _zero_rows(rows0, ZRB, D)
  zsrc = rows0.at[pl.ds(0, ZRB)]
  for t in range(NZB):
    pltpu.sync_copy(zsrc, acc_sh.at[pl.ds(s * RPS + t * ZRB, ZRB)])
  plsc.subcore_barrier()

  def win_slice(ref, w):
    off = pl.multiple_of(w * WS, WS)
    return ref.at[wid, pl.ds(off, WS)]

  pltpu.sync_copy(win_slice(src_hbm, 0), sidx0)
  pltpu.sync_copy(win_slice(dst_hbm, 0), didx0)
  pltpu.async_copy(win_slice(src_hbm, 1), sidx1, isem1)
  pltpu.async_copy(win_slice(dst_hbm, 1), didx1, isem1)
  pltpu.async_copy(cur_hbm.at[sidx0.at[0]], rows0, gsem0)
  pltpu.async_copy(cur_hbm.at[sidx0.at[1]], rows1, gsem1)

  def do_window(w, wb, base_b):
    for t in range(WS):
      b = (base_b + t) % 3
      bn = (base_b + t + 2) % 3
      if t < WS - 2:
        pltpu.async_copy(cur_hbm.at[sidx[wb].at[t + 2]], rows[bn], gsems[bn])
      elif t == WS - 2:
        nxt_first = (w + 1) * WS

        @pl.when(nxt_first < NCHB)
        def _():
          pltpu.make_async_copy(win_slice(src_hbm, w + 1), sidx[1 - wb],
                                isems[1 - wb]).wait()
          pltpu.make_async_copy(win_slice(dst_hbm, w + 1), didx[1 - wb],
                                isems[1 - wb]).wait()
          pltpu.async_copy(cur_hbm.at[sidx[1 - wb].at[0]], rows[bn],
                           gsems[bn])
      else:
        nxt_second = (w + 1) * WS + 1

        @pl.when(nxt_second < NCHB)
        def _():
          pltpu.async_copy(cur_hbm.at[sidx[1 - wb].at[1]], rows[bn],
                           gsems[bn])
      pltpu.make_async_copy(cur_hbm.at[sidx[wb].at[t]], rows[b],
                            gsems[b]).wait()
      pltpu.sync_copy(rows[b], acc_sh.at[didx[wb].at[t]], add=True)

    @pl.when(w + 2 < NWIN)
    def _():
      pltpu.async_copy(win_slice(src_hbm, w + 2), sidx[wb], isems[wb])
      pltpu.async_copy(win_slice(dst_hbm, w + 2), didx[wb], isems[wb])

  def group(i, _):
    for k in range(6):
      do_window(i * 6 + k, k % 2, (2 * k) % 3)
    return 0

  lax.fori_loop(0, (NWIN - 1) // 6, group, 0)
  do_window(NWIN - 1, 0, 0)
  plsc.subcore_barrier()

  pltpu.sync_copy(acc_sh.at[pl.ds(s * RPS, RPS)],
                  out_hbm.at[c, pl.ds(s * RPS, RPS)])


NB = 10
BR = N // NB


def _msg_init_tc(nf, seg, Wn, We, bn, be, im_out, cur_out):
  sfull = seg[0] + seg[1]
  s = sfull[:, :DE]
  deg = sfull[:, DE:DE + 1]
  im = (jnp.dot(nf[...], Wn[...], preferred_element_type=jnp.float32)
        + jnp.dot(s, We[...], preferred_element_type=jnp.float32)
        + bn[...] + deg * be[...])
  im_out[...] = im
  cur_out[...] = jnp.maximum(im, 0.0)


def _round_tc(p, im, Wc, bc, cur_out):
  npool = p[0] + p[1]
  nl = jnp.dot(npool, Wc[...], preferred_element_type=jnp.float32)
  cur_out[...] = jnp.maximum(nl + bc[...] + im[...], 0.0)


def _round_final_tc(p, im, Wc, bc, gids, Wfp, bfp, y_out):
  i = pl.program_id(0)
  npool = p[0] + p[1]
  nl = jnp.dot(npool, Wc[...], preferred_element_type=jnp.float32)
  cur = jnp.maximum(nl + bc[...] + im[...], 0.0)
  r = jnp.maximum(
      jnp.dot(cur, Wfp[...], preferred_element_type=jnp.float32)
      + bfp[...], 0.0)
  ids = gids[0]
  onehot = (lax.broadcasted_iota(jnp.int32, (G, BR), 0) == ids).astype(
      jnp.float32)
  contrib = jnp.dot(onehot, r, preferred_element_type=jnp.float32)

  @pl.when(i == 0)
  def _():
    y_out[...] = jnp.zeros_like(y_out)

  y_out[...] += contrib


def kernel(node_feat, edge_feat, edge_index, graph_ids, Wn, bn, We, be,
           Wc0, bc0, Wc1, bc1, Wc2, bc2, Wfp, bfp):
  src = edge_index[0].reshape(NW, NCHB, KB)
  dst = edge_index[1].reshape(NW, NCHB, KB)
  dst_a = edge_index[1].reshape(NW, NCHA, KA)
  ef = edge_feat.reshape(NW, NCHA, KA, DE)
  gids3 = graph_ids.reshape(NB, 1, BR)
  bn2 = bn.reshape(1, D)
  be2 = be.reshape(1, D)
  bfp2 = bfp.reshape(1, D)

  seg32 = _edge_pool_sc(ef, dst_a)

  row_spec = pl.BlockSpec((BR, D), lambda i: (i, 0))
  full2 = lambda shape: pl.BlockSpec(shape, lambda i: tuple(0 for _ in shape))

  im, cur = pl.pallas_call(
      _msg_init_tc,
      grid=(NB,),
      in_specs=[
          row_spec,
          pl.BlockSpec((NC, BR, DA), lambda i: (0, i, 0)),
          full2((D, D)), full2((DE, D)), full2((1, D)), full2((1, D)),
      ],
      out_specs=[row_spec, row_spec],
      out_shape=[
          jax.ShapeDtypeStruct((N, D), jnp.float32),
          jax.ShapeDtypeStruct((N, D), jnp.float32),
      ],
  )(node_feat, seg32, Wn, We, bn2, be2)

  round_call = pl.pallas_call(
      _round_tc,
      grid=(NB,),
      in_specs=[
          pl.BlockSpec((NC, BR, D), lambda i: (0, i, 0)),
          row_spec,
          full2((D, D)), full2((1, D)),
      ],
      out_specs=row_spec,
      out_shape=jax.ShapeDtypeStruct((N, D), jnp.float32),
  )

  for Wc, bc in ((Wc0, bc0), (Wc1, bc1)):
    partials = _neigh_pool_sc(cur, src, dst)
    cur = round_call(partials, im, Wc, bc.reshape(1, D))

  partials = _neigh_pool_sc(cur, src, dst)
  y = pl.pallas_call(
      _round_final_tc,
      grid=(NB,),
      in_specs=[
          pl.BlockSpec((NC, BR, D), lambda i: (0, i, 0)),
          row_spec,
          full2((D, D)), full2((1, D)),
          pl.BlockSpec((1, 1, BR), lambda i: (i, 0, 0)),
          full2((D, D)), full2((1, D)),
      ],
      out_specs=pl.BlockSpec((G, D), lambda i: (0, 0)),
      out_shape=jax.ShapeDtypeStruct((G, D), jnp.float32),
  )(partials, im, Wc2, bc2.reshape(1, D), gids3, Wfp, bfp2)

  return y


# speedup vs baseline: 8.2479x; 1.0003x over previous
import functools
from functools import partial

import jax
import jax.numpy as jnp
from jax import lax
from jax.experimental import pallas as pl
from jax.experimental.pallas import tpu as pltpu
from jax.experimental.pallas import tpu_sc as plsc

N = 10000
E = 320000
D = 128
DE = 16
G = 64

NC = 2
NS = 16
NW = NC * NS
EPW = E // NW
KA = 100
NCHA = EPW // KA
KB = 50
NCHB = EPW // KB
WS = 8
NWIN = NCHB // WS
NP = 10240
RPS = NP // NS
ZR = 80
NZ = RPS // ZR
ZRB = 40
NZB = RPS // ZRB

_mesh = plsc.VectorSubcoreMesh(
    core_axis_name="c", subcore_axis_name="s", num_cores=NC, num_subcores=NS)


def _zero_rows(zbuf, nrows, ncols):
  z = jnp.zeros((16,), jnp.float32)

  def body(i, _):
    for k in range(ncols // 16):
      zbuf[i, pl.ds(k * 16, 16)] = z
    return 0

  lax.fori_loop(0, nrows, body, 0)


def _ones_rows(obuf, nrows, ncols):
  o = jnp.ones((16,), jnp.float32)

  def body(i, _):
    for k in range(ncols // 16):
      obuf[i, pl.ds(k * 16, 16)] = o
    return 0

  lax.fori_loop(0, nrows, body, 0)


DA = 2 * DE


@partial(
    pl.kernel,
    out_type=jax.ShapeDtypeStruct((NC, NP, DA), jnp.float32),
    mesh=_mesh,
    compiler_params=pltpu.CompilerParams(use_tc_tiling_on_sc=False,
                                         skip_device_barrier=True),
    scratch_types=[
        pltpu.VMEM((NCHA, KA), jnp.int32),
        pltpu.VMEM((KA, DA), jnp.float32),
        pltpu.VMEM((KA, DA), jnp.float32),
        pltpu.VMEM((KA, DA), jnp.float32),
        pltpu.VMEM_SHARED((NP, DA), jnp.float32),
        pltpu.SemaphoreType.DMA,
        pltpu.SemaphoreType.DMA,
        pltpu.SemaphoreType.DMA,
    ],
)
def _edge_pool_sc(ef_hbm, dst_hbm, out_hbm, idx_v, r0, r1, r2, acc_sh,
                  g0, g1, g2):
  c = lax.axis_index("c")
  s = lax.axis_index("s")
  wid = c * NS + s
  rows = (r0, r1, r2)
  gsems = (g0, g1, g2)

  _zero_rows(r0, ZR, DA)
  zsrc = r0.at[pl.ds(0, ZR)]
  for t in range(NZ):
    pltpu.sync_copy(zsrc, acc_sh.at[pl.ds(s * RPS + t * ZR, ZR)])
  plsc.subcore_barrier()

  onev = jnp.where(lax.iota(jnp.int32, 16) == 0, 1.0, 0.0).astype(jnp.float32)

  def initrows(i, _):
    for rb in rows:
      rb[i, pl.ds(DE, 16)] = onev
    return 0

  lax.fori_loop(0, KA, initrows, 0)

  pltpu.sync_copy(dst_hbm.at[wid], idx_v)

  def feat_dst(rb):
    return rb.at[:, pl.ds(0, DE)]

  pltpu.async_copy(ef_hbm.at[wid, 0], feat_dst(r0), g0)
  pltpu.async_copy(ef_hbm.at[wid, 1], feat_dst(r1), g1)

  def body(i, _):
    for b in range(3):
      j = i * 3 + b

      @pl.when(j + 2 < NCHA)
      def _():
        pltpu.async_copy(ef_hbm.at[wid, j + 2], feat_dst(rows[(b + 2) % 3]),
                         gsems[(b + 2) % 3])

      pltpu.make_async_copy(ef_hbm.at[wid, j], feat_dst(rows[b]),
                            gsems[b]).wait()
      pltpu.sync_copy(rows[b], acc_sh.at[idx_v.at[j]], add=True)
    return 0

  lax.fori_loop(0, NCHA // 3, body, 0)
  jt = NCHA - 1
  pltpu.make_async_copy(ef_hbm.at[wid, jt], feat_dst(r0), g0).wait()
  pltpu.sync_copy(r0, acc_sh.at[idx_v.at[jt]], add=True)
  plsc.subcore_barrier()

  pltpu.sync_copy(acc_sh.at[pl.ds(s * RPS, RPS)],
                  out_hbm.at[c, pl.ds(s * RPS, RPS)])


@partial(
    pl.kernel,
    out_type=jax.ShapeDtypeStruct((NC, NP, D), jnp.float32),
    mesh=_mesh,
    compiler_params=pltpu.CompilerParams(skip_device_barrier=True),
    scratch_types=[
        pltpu.VMEM((WS, KB), jnp.int32),
        pltpu.VMEM((WS, KB), jnp.int32),
        pltpu.VMEM((WS, KB), jnp.int32),
        pltpu.VMEM((WS, KB), jnp.int32),
        pltpu.VMEM((KB, D), jnp.float32),
        pltpu.VMEM((KB, D), jnp.float32),
        pltpu.VMEM((KB, D), jnp.float32),
        pltpu.VMEM_SHARED((NP, D), jnp.float32),
        pltpu.SemaphoreType.DMA,
        pltpu.SemaphoreType.DMA,
        pltpu.SemaphoreType.DMA,
        pltpu.SemaphoreType.DMA,
        pltpu.SemaphoreType.DMA,
    ],
)
def _neigh_pool_sc(cur_hbm, src_hbm, dst_hbm, out_hbm, sidx0, sidx1, didx0,
                   didx1, rows0, rows1, rows2, acc_sh, gsem0, gsem1, gsem2,
                   isem0, isem1):
  c = lax.axis_index("c")
  s = lax.axis_index("s")
  wid = c * NS + s
  rows = (rows0, rows1, rows2)
  gsems = (gsem0, gsem1, gsem2)
  sidx = (sidx0, sidx1)
  didx = (didx0, didx1)
  isems = (isem0, isem1)

  _zero_rows(rows0, ZRB, D)
  zsrc = rows0.at[pl.ds(0, ZRB)]
  for t in range(NZB):
    pltpu.sync_copy(zsrc, acc_sh.at[pl.ds(s * RPS + t * ZRB, ZRB)])
  plsc.subcore_barrier()

  def win_slice(ref, w):
    off = pl.multiple_of(w * WS, WS)
    return ref.at[wid, pl.ds(off, WS)]

  pltpu.sync_copy(win_slice(src_hbm, 0), sidx0)
  pltpu.sync_copy(win_slice(dst_hbm, 0), didx0)
  pltpu.async_copy(win_slice(src_hbm, 1), sidx1, isem1)
  pltpu.async_copy(win_slice(dst_hbm, 1), didx1, isem1)
  pltpu.async_copy(cur_hbm.at[sidx0.at[0]], rows0, gsem0)
  pltpu.async_copy(cur_hbm.at[sidx0.at[1]], rows1, gsem1)

  def do_window(w, wb, base_b):
    for t in range(WS):
      b = (base_b + t) % 3
      bn = (base_b + t + 2) % 3
      if t < WS - 2:
        pltpu.async_copy(cur_hbm.at[sidx[wb].at[t + 2]], rows[bn], gsems[bn])
      elif t == WS - 2:
        nxt_first = (w + 1) * WS

        @pl.when(nxt_first < NCHB)
        def _():
          pltpu.make_async_copy(win_slice(src_hbm, w + 1), sidx[1 - wb],
                                isems[1 - wb]).wait()
          pltpu.make_async_copy(win_slice(dst_hbm, w + 1), didx[1 - wb],
                                isems[1 - wb]).wait()
          pltpu.async_copy(cur_hbm.at[sidx[1 - wb].at[0]], rows[bn],
                           gsems[bn])
      else:
        nxt_second = (w + 1) * WS + 1

        @pl.when(nxt_second < NCHB)
        def _():
          pltpu.async_copy(cur_hbm.at[sidx[1 - wb].at[1]], rows[bn],
                           gsems[bn])
      pltpu.make_async_copy(cur_hbm.at[sidx[wb].at[t]], rows[b],
                            gsems[b]).wait()
      pltpu.sync_copy(rows[b], acc_sh.at[didx[wb].at[t]], add=True)

    @pl.when(w + 2 < NWIN)
    def _():
      pltpu.async_copy(win_slice(src_hbm, w + 2), sidx[wb], isems[wb])
      pltpu.async_copy(win_slice(dst_hbm, w + 2), didx[wb], isems[wb])

  def group(i, _):
    for k in range(6):
      do_window(i * 6 + k, k % 2, (2 * k) % 3)
    return 0

  lax.fori_loop(0, (NWIN - 1) // 6, group, 0)
  do_window(NWIN - 1, 0, 0)
  plsc.subcore_barrier()

  pltpu.sync_copy(acc_sh.at[pl.ds(s * RPS, RPS)],
                  out_hbm.at[c, pl.ds(s * RPS, RPS)])


NB = 10
BR = N // NB


def _msg_init_tc(nf, seg, Wn, We, bn, be, im_out, cur_out):
  sfull = seg[0] + seg[1]
  s = sfull[:, :DE]
  deg = sfull[:, DE:DE + 1]
  im = (jnp.dot(nf[...], Wn[...], preferred_element_type=jnp.float32)
        + jnp.dot(s, We[...], preferred_element_type=jnp.float32)
        + bn[...] + deg * be[...])
  im_out[...] = im
  cur_out[...] = jnp.maximum(im, 0.0)


def _round_tc(p, im, Wc, bc, cur_out):
  npool = p[0] + p[1]
  nl = jnp.dot(npool, Wc[...], preferred_element_type=jnp.float32)
  cur_out[...] = jnp.maximum(nl + bc[...] + im[...], 0.0)


def _round_final_tc(p, im, Wc, bc, gids, Wfp, bfp, y_out):
  i = pl.program_id(0)
  npool = p[0] + p[1]
  nl = jnp.dot(npool, Wc[...], preferred_element_type=jnp.float32)
  cur = jnp.maximum(nl + bc[...] + im[...], 0.0)
  r = jnp.maximum(
      jnp.dot(cur, Wfp[...], preferred_element_type=jnp.float32)
      + bfp[...], 0.0)
  ids = gids[0]
  onehot = (lax.broadcasted_iota(jnp.int32, (G, BR), 0) == ids).astype(
      jnp.float32)
  contrib = jnp.dot(onehot, r, preferred_element_type=jnp.float32)

  @pl.when(i == 0)
  def _():
    y_out[...] = jnp.zeros_like(y_out)

  y_out[...] += contrib


def kernel(node_feat, edge_feat, edge_index, graph_ids, Wn, bn, We, be,
           Wc0, bc0, Wc1, bc1, Wc2, bc2, Wfp, bfp):
  src = edge_index[0].reshape(NW, NCHB, KB)
  dst = edge_index[1].reshape(NW, NCHB, KB)
  dst_a = edge_index[1].reshape(NW, NCHA, KA)
  ef = edge_feat.reshape(NW, NCHA, KA, DE)
  gids3 = graph_ids.reshape(NB, 1, BR)
  bn2 = bn.reshape(1, D)
  be2 = be.reshape(1, D)
  bfp2 = bfp.reshape(1, D)

  seg32 = _edge_pool_sc(ef, dst_a)

  row_spec = pl.BlockSpec((BR, D), lambda i: (i, 0))
  full2 = lambda shape: pl.BlockSpec(shape, lambda i: tuple(0 for _ in shape))

  im, cur = pl.pallas_call(
      _msg_init_tc,
      grid=(NB,),
      in_specs=[
          row_spec,
          pl.BlockSpec((NC, BR, DA), lambda i: (0, i, 0)),
          full2((D, D)), full2((DE, D)), full2((1, D)), full2((1, D)),
      ],
      out_specs=[row_spec, row_spec],
      out_shape=[
          jax.ShapeDtypeStruct((N, D), jnp.float32),
          jax.ShapeDtypeStruct((N, D), jnp.float32),
      ],
  )(node_feat, seg32, Wn, We, bn2, be2)

  round_call = pl.pallas_call(
      _round_tc,
      grid=(NB,),
      in_specs=[
          pl.BlockSpec((NC, BR, D), lambda i: (0, i, 0)),
          row_spec,
          full2((D, D)), full2((1, D)),
      ],
      out_specs=row_spec,
      out_shape=jax.ShapeDtypeStruct((N, D), jnp.float32),
  )

  for Wc, bc in ((Wc0, bc0), (Wc1, bc1)):
    partials = _neigh_pool_sc(cur, src, dst)
    cur = round_call(partials, im, Wc, bc.reshape(1, D))

  partials = _neigh_pool_sc(cur, src, dst)
  y = pl.pallas_call(
      _round_final_tc,
      grid=(NB,),
      in_specs=[
          pl.BlockSpec((NC, BR, D), lambda i: (0, i, 0)),
          row_spec,
          full2((D, D)), full2((1, D)),
          pl.BlockSpec((1, 1, BR), lambda i: (i, 0, 0)),
          full2((D, D)), full2((1, D)),
      ],
      out_specs=pl.BlockSpec((G, D), lambda i: (0, 0)),
      out_shape=jax.ShapeDtypeStruct((G, D), jnp.float32),
  )(partials, im, Wc2, bc2.reshape(1, D), gids3, Wfp, bfp2)

  return y
